# Initial kernel scaffold; baseline (speedup 1.0000x reference)
import functools

import jax
import jax.numpy as jnp
from jax import lax
from jax.experimental import pallas as pl
from jax.experimental.pallas import tpu as pltpu
from jax.experimental.pallas import tpu_sc as plsc

B = 4
N = 8192
NP = 2048
K = 16
C = 128
C2 = 256


_SUB = 8
_LANES = N // _SUB


def _fps_body(cpad_ref, ct_ref, newc_ref, newcp_ref, dists_ref):
    dists_ref[...] = jnp.full((B, _SUB, _LANES), 1e10, jnp.float32)
    row_iota = lax.broadcasted_iota(jnp.int32, (_SUB, _LANES), 0)
    col_iota = lax.broadcasted_iota(jnp.int32, (_SUB, _LANES), 1)
    lin = row_iota * _LANES + col_iota

    def body(i, fs):
        nfs = []
        for b in range(B):
            f = fs[b]
            c = cpad_ref[b, pl.ds(f, 1), :]
            newcp_ref[b, pl.ds(i, 1), :] = c
            newc_ref[b, pl.ds(i, 1), :] = c[:, 0:3]
            cx = jnp.max(c[:, 0:1])
            cy = jnp.max(c[:, 1:2])
            cz = jnp.max(c[:, 2:3])
            dx = ct_ref[b, 0] - cx
            dy = ct_ref[b, 1] - cy
            dz = ct_ref[b, 2] - cz
            d = (dx * dx + dy * dy) + dz * dz
            nd = jnp.minimum(dists_ref[b], d)
            dists_ref[b] = nd
            m = jnp.max(nd)
            cand = jnp.where(nd == m, lin, N)
            nfs.append(jnp.min(cand))
        return tuple(nfs)

    zero = jnp.int32(0)
    lax.fori_loop(0, NP, body, (zero, zero, zero, zero))


def _fps_call(cpad, ct):
    return pl.pallas_call(
        _fps_body,
        out_shape=(
            jax.ShapeDtypeStruct((B, NP, 3), jnp.float32),
            jax.ShapeDtypeStruct((B, NP, 8), jnp.float32),
        ),
        scratch_shapes=[pltpu.VMEM((B, _SUB, _LANES), jnp.float32)],
    )(cpad, ct)


_RB = 128


def _knn_body(xp_ref, idx_ref):
    bi = pl.program_id(0)
    ri = pl.program_id(1)
    xb = xp_ref[0]
    rows = xb[pl.ds(ri * _RB, _RB), :]
    nt = (((1,), (1,)), ((), ()))
    inner = lax.dot_general(rows, xb, nt, preferred_element_type=jnp.float32)
    xb2 = xb * xb
    rows2 = rows * rows
    sqc = jnp.sum(rows2, axis=1, keepdims=True)
    ones = jnp.ones((1, 8), jnp.float32)
    sqr = lax.dot_general(ones, xb2, nt, preferred_element_type=jnp.float32)
    dist = (sqc + sqr) - 2.0 * inner
    lane = lax.broadcasted_iota(jnp.int32, (_RB, NP), 1)
    big = jnp.float32(3.0e38)
    for r in range(K):
        m = jnp.min(dist, axis=1, keepdims=True)
        cand = jnp.where(dist == m, lane, NP)
        am = jnp.min(cand, axis=1, keepdims=True)
        idx_ref[0, :, r : r + 1] = am + bi * NP
        dist = jnp.where(lane == am, big, dist)


def _knn_call(newcp):
    grid = (B, NP // _RB)
    return pl.pallas_call(
        _knn_body,
        grid=grid,
        in_specs=[pl.BlockSpec((1, NP, 8), lambda b, r: (b, 0, 0))],
        out_specs=pl.BlockSpec((1, _RB, K), lambda b, r: (b, r, 0)),
        out_shape=jax.ShapeDtypeStruct((B, NP, K), jnp.int32),
    )(newcp)


_MB = 1024


def _mm_body(x_ref, w_ref, y_ref):
    y_ref[...] = jnp.dot(x_ref[...], w_ref[...], preferred_element_type=jnp.float32)


def _mm_call(x, w):
    grid = (x.shape[0] // _MB,)
    return pl.pallas_call(
        _mm_body,
        grid=grid,
        in_specs=[
            pl.BlockSpec((_MB, C), lambda i: (i, 0)),
            pl.BlockSpec((C, C2), lambda i: (0, 0)),
        ],
        out_specs=pl.BlockSpec((_MB, C2), lambda i: (i, 0)),
        out_shape=jax.ShapeDtypeStruct((x.shape[0], C2), jnp.float32),
    )(x, w)


_NW = 32
_RPW = (B * NP) // _NW


def _gather_max_body(y_hbm, idx_hbm, b1_hbm, out_hbm, idx_v, rows_v, b1_v, out_v, sem):
    wid = lax.axis_index("s") * 2 + lax.axis_index("c")
    base = wid * _RPW
    pltpu.sync_copy(b1_hbm, b1_v)

    def body(r, carry):
        row = base + r
        pltpu.sync_copy(idx_hbm.at[row], idx_v)
        pltpu.async_copy(y_hbm.at[idx_v], rows_v, sem).wait()
        for c in range(C2 // 16):
            sl = pl.ds(c * 16, 16)
            m = rows_v[0, sl]
            for j in range(1, K):
                m = jnp.maximum(m, rows_v[j, sl])
            out_v[sl] = jnp.maximum(m + b1_v[sl], 0.0)
        pltpu.sync_copy(out_v, out_hbm.at[row])
        return carry

    lax.fori_loop(0, _RPW, body, jnp.int32(0))


def _gather_max_call(y, idx, b1):
    mesh = plsc.VectorSubcoreMesh(core_axis_name="c", subcore_axis_name="s")
    return pl.kernel(
        _gather_max_body,
        out_type=jax.ShapeDtypeStruct((B * NP, C2), jnp.float32),
        mesh=mesh,
        scratch_types=[
            pltpu.VMEM((K,), jnp.int32),
            pltpu.VMEM((K, C2), jnp.float32),
            pltpu.VMEM((C2,), jnp.float32),
            pltpu.VMEM((C2,), jnp.float32),
            pltpu.SemaphoreType.DMA,
        ],
    )(y, idx, b1)


def kernel(coords, features, W1, b1):
    coords = lax.stop_gradient(coords)
    cpad = jnp.pad(coords, ((0, 0), (0, 0), (0, 5)))
    ct = coords.transpose(0, 2, 1).reshape(B, 3, _SUB, _LANES)
    newc, newcp = _fps_call(cpad, ct)
    idx = _knn_call(newcp).reshape(B * NP, K)
    y = _mm_call(features[:, :NP, :].reshape(B * NP, C), W1)
    pooled = _gather_max_call(y, idx, b1).reshape(B, NP, C2)
    return (newc, pooled)


# speedup vs baseline: 7.7053x; 7.7053x over previous
import functools

import jax
import jax.numpy as jnp
from jax import lax
from jax.experimental import pallas as pl
from jax.experimental.pallas import tpu as pltpu
from jax.experimental.pallas import tpu_sc as plsc

B = 4
N = 8192
NP = 2048
K = 16
C = 128
C2 = 256


_SUB = 8
_LANES = N // _SUB


def _fps_body(cpad_ref, ct_ref, newc_ref, newcp_ref, newsq_ref, dists_ref):
    dists_ref[...] = jnp.full((B, _SUB, _LANES), 1e10, jnp.float32)
    row_iota = lax.broadcasted_iota(jnp.int32, (_SUB, _LANES), 0)
    col_iota = lax.broadcasted_iota(jnp.int32, (_SUB, _LANES), 1)
    lin = row_iota * _LANES + col_iota

    def body(i, fs):
        nfs = []
        for b in range(B):
            f = fs[b]
            c = cpad_ref[b, pl.ds(f, 1), :]
            newcp_ref[b, pl.ds(i, 1), :] = c
            newc_ref[b, pl.ds(i, 1), :] = c[:, 0:3]
            x2 = c[:, 0:1] * c[:, 0:1]
            y2 = c[:, 1:2] * c[:, 1:2]
            z2 = c[:, 2:3] * c[:, 2:3]
            newsq_ref[b, pl.ds(i, 1), :] = (x2 + y2) + z2
            cx = jnp.max(c[:, 0:1])
            cy = jnp.max(c[:, 1:2])
            cz = jnp.max(c[:, 2:3])
            dx = ct_ref[b, 0] - cx
            dy = ct_ref[b, 1] - cy
            dz = ct_ref[b, 2] - cz
            d = (dx * dx + dy * dy) + dz * dz
            nd = jnp.minimum(dists_ref[b], d)
            dists_ref[b] = nd
            m = jnp.max(nd)
            cand = jnp.where(nd == m, lin, N)
            nfs.append(jnp.min(cand))
        return tuple(nfs)

    zero = jnp.int32(0)
    lax.fori_loop(0, NP, body, (zero, zero, zero, zero))


def _fps_call(cpad, ct):
    return pl.pallas_call(
        _fps_body,
        out_shape=(
            jax.ShapeDtypeStruct((B, NP, 3), jnp.float32),
            jax.ShapeDtypeStruct((B, NP, 8), jnp.float32),
            jax.ShapeDtypeStruct((B, NP, 1), jnp.float32),
        ),
        scratch_shapes=[pltpu.VMEM((B, _SUB, _LANES), jnp.float32)],
    )(cpad, ct)


_RB = 128


def _knn_body(xp_ref, sqc_ref, sqr_ref, idx_ref):
    bi = pl.program_id(0)
    ri = pl.program_id(1)
    xb = xp_ref[0].astype(jnp.bfloat16)
    rows = xp_ref[0, pl.ds(ri * _RB, _RB), :].astype(jnp.bfloat16)
    nt = (((1,), (1,)), ((), ()))
    inner = lax.dot_general(rows, xb, nt, preferred_element_type=jnp.float32)
    sqc = sqc_ref[0, pl.ds(ri * _RB, _RB), :]
    sqr = sqr_ref[0]
    dist = (sqc + sqr) - 2.0 * inner
    lane = lax.broadcasted_iota(jnp.int32, (_RB, NP), 1)
    big = jnp.float32(3.0e38)
    for r in range(K):
        m = jnp.min(dist, axis=1, keepdims=True)
        cand = jnp.where(dist == m, lane, NP)
        am = jnp.min(cand, axis=1, keepdims=True)
        idx_ref[0, :, r : r + 1] = am + bi * NP
        dist = jnp.where(lane == am, big, dist)


def _knn_call(newcp, newsq):
    grid = (B, NP // _RB)
    sqr = newsq.reshape(B, 1, NP)
    return pl.pallas_call(
        _knn_body,
        grid=grid,
        in_specs=[
            pl.BlockSpec((1, NP, 8), lambda b, r: (b, 0, 0)),
            pl.BlockSpec((1, NP, 1), lambda b, r: (b, 0, 0)),
            pl.BlockSpec((1, 1, NP), lambda b, r: (b, 0, 0)),
        ],
        out_specs=pl.BlockSpec((1, _RB, K), lambda b, r: (b, r, 0)),
        out_shape=jax.ShapeDtypeStruct((B, NP, K), jnp.int32),
    )(newcp, newsq, sqr)


_MB = 1024


def _mm_body(x_ref, w_ref, y_ref):
    y_ref[...] = jnp.dot(x_ref[...], w_ref[...], preferred_element_type=jnp.float32)


def _mm_call(x, w):
    grid = (x.shape[0] // _MB,)
    return pl.pallas_call(
        _mm_body,
        grid=grid,
        in_specs=[
            pl.BlockSpec((_MB, C), lambda i: (i, 0)),
            pl.BlockSpec((C, C2), lambda i: (0, 0)),
        ],
        out_specs=pl.BlockSpec((_MB, C2), lambda i: (i, 0)),
        out_shape=jax.ShapeDtypeStruct((x.shape[0], C2), jnp.float32),
    )(x, w)


_NW = 32
_RPW = (B * NP) // _NW


def _gather_max_body(y_hbm, idx_hbm, b1_hbm, out_hbm, idx_v, rows_v, b1_v, out_v, sem):
    wid = lax.axis_index("s") * 2 + lax.axis_index("c")
    base = wid * _RPW
    pltpu.sync_copy(b1_hbm, b1_v)

    def body(r, carry):
        row = base + r
        pltpu.sync_copy(idx_hbm.at[row], idx_v)
        pltpu.async_copy(y_hbm.at[idx_v], rows_v, sem).wait()
        for c in range(C2 // 16):
            sl = pl.ds(c * 16, 16)
            m = rows_v[0, sl]
            for j in range(1, K):
                m = jnp.maximum(m, rows_v[j, sl])
            out_v[sl] = jnp.maximum(m + b1_v[sl], 0.0)
        pltpu.sync_copy(out_v, out_hbm.at[row])
        return carry

    lax.fori_loop(0, _RPW, body, jnp.int32(0))


def _gather_max_call(y, idx, b1):
    mesh = plsc.VectorSubcoreMesh(core_axis_name="c", subcore_axis_name="s")
    return pl.kernel(
        _gather_max_body,
        out_type=jax.ShapeDtypeStruct((B * NP, C2), jnp.float32),
        mesh=mesh,
        scratch_types=[
            pltpu.VMEM((K,), jnp.int32),
            pltpu.VMEM((K, C2), jnp.float32),
            pltpu.VMEM((C2,), jnp.float32),
            pltpu.VMEM((C2,), jnp.float32),
            pltpu.SemaphoreType.DMA,
        ],
    )(y, idx, b1)


def kernel(coords, features, W1, b1):
    coords = lax.stop_gradient(coords)
    cpad = jnp.pad(coords, ((0, 0), (0, 0), (0, 5)))
    ct = coords.transpose(0, 2, 1).reshape(B, 3, _SUB, _LANES)
    newc, newcp, newsq = _fps_call(cpad, ct)
    idx = _knn_call(newcp, newsq).reshape(B * NP, K)
    y = _mm_call(features[:, :NP, :].reshape(B * NP, C), W1)
    pooled = _gather_max_call(y, idx, b1).reshape(B, NP, C2)
    return (newc, pooled)
